# Initial kernel scaffold; baseline (speedup 1.0000x reference)
import functools

import jax
import jax.numpy as jnp
from jax import lax
from jax.experimental import pallas as pl
from jax.experimental.pallas import tpu as pltpu
from jax.experimental.pallas import tpu_sc as plsc

N = 10000
E = 320000
D = 128

NC = 2
NS = 16
NW = NC * NS
EPW = E // NW
CH = 80
NCH = EPW // CH
RPT = N // NS
ZR = 125


def _bcast_lane(vec16, r):
    idx = jnp.full((16,), r, dtype=jnp.int32)
    return jnp.take(vec16, idx, mode="promise_in_bounds")


def _make_agg(with_deg: bool):
    mesh = plsc.VectorSubcoreMesh(core_axis_name="c", subcore_axis_name="s")
    out_type = [jax.ShapeDtypeStruct((NC, N, D), jnp.float32)]
    if with_deg:
        out_type.append(jax.ShapeDtypeStruct((NC, N, 16), jnp.float32))
    scratch = [
        pltpu.VMEM((CH,), jnp.int32),
        pltpu.VMEM((CH,), jnp.int32),
        pltpu.VMEM((CH,), jnp.float32),
        pltpu.VMEM((CH, D), jnp.float32),
        pltpu.VMEM((ZR, D), jnp.float32),
        pltpu.VMEM_SHARED((N, D), jnp.float32),
        pltpu.SemaphoreType.DMA,
    ]
    if with_deg:
        scratch += [
            pltpu.VMEM((CH, 16), jnp.float32),
            pltpu.VMEM((RPT, 16), jnp.float32),
            pltpu.VMEM_SHARED((N, 16), jnp.float32),
        ]

    def body(x_hbm, src_hbm, dst_hbm, ew_hbm, *rest):
        if with_deg:
            (out_sum, out_deg, idx_v, dst_v, ew_v, rows_v, zbuf, acc_sh, sem,
             ones_v, zdeg, deg_sh) = rest
        else:
            (out_sum, idx_v, dst_v, ew_v, rows_v, zbuf, acc_sh, sem) = rest
        cid = lax.axis_index("c")
        sid = lax.axis_index("s")
        wid = sid * NC + cid
        rbase = sid * RPT

        def zrow(r, _):
            for c8 in range(D // 16):
                zbuf[r, pl.ds(c8 * 16, 16)] = jnp.zeros((16,), jnp.float32)
            return 0
        lax.fori_loop(0, ZR, zrow, 0)
        for k in range(RPT // ZR):
            pltpu.sync_copy(zbuf, acc_sh.at[pl.ds(rbase + k * ZR, ZR), :])
        if with_deg:
            def zdrow(r, _):
                zdeg[r, pl.ds(0, 16)] = jnp.zeros((16,), jnp.float32)
                return 0
            lax.fori_loop(0, RPT, zdrow, 0)

            def orow(r, _):
                ones_v[r, pl.ds(0, 16)] = jnp.ones((16,), jnp.float32)
                return 0
            lax.fori_loop(0, CH, orow, 0)
            pltpu.sync_copy(zdeg, deg_sh.at[pl.ds(rbase, RPT), :])
        plsc.subcore_barrier()

        def chunk(i, _):
            base = wid * EPW + i * CH
            pltpu.sync_copy(src_hbm.at[pl.ds(base, CH)], idx_v)
            pltpu.async_copy(x_hbm.at[idx_v], rows_v, sem).wait()
            pltpu.sync_copy(dst_hbm.at[pl.ds(base, CH)], dst_v)
            pltpu.sync_copy(ew_hbm.at[pl.ds(base, CH)], ew_v)

            def grp(j, _):
                ew16 = ew_v[pl.ds(j * 16, 16)]
                for r in range(16):
                    w = _bcast_lane(ew16, r)
                    e = j * 16 + r
                    for c8 in range(D // 16):
                        sl = pl.ds(c8 * 16, 16)
                        rows_v[e, sl] = rows_v[e, sl] * w
                return 0
            lax.fori_loop(0, CH // 16, grp, 0)

            pltpu.sync_copy(rows_v, acc_sh.at[dst_v], add=True)
            if with_deg:
                pltpu.sync_copy(ones_v, deg_sh.at[dst_v], add=True)
            return 0
        lax.fori_loop(0, NCH, chunk, 0)

        plsc.subcore_barrier()
        pltpu.sync_copy(acc_sh.at[pl.ds(rbase, RPT), :],
                        out_sum.at[cid, pl.ds(rbase, RPT), :])
        if with_deg:
            pltpu.sync_copy(deg_sh.at[pl.ds(rbase, RPT), :],
                            out_deg.at[cid, pl.ds(rbase, RPT), :])

    return pl.kernel(body, out_type=tuple(out_type), mesh=mesh,
                     scratch_types=scratch)


_agg_deg = _make_agg(with_deg=True)
_agg = _make_agg(with_deg=False)


_RB = 1000


def _sage_dense_body(x_ref, acc_ref, deg_ref, wst_ref, wnt_ref, b_ref, o_ref):
    x = x_ref[...]
    acc = acc_ref[0] + acc_ref[1]
    deg = deg_ref[0, :, 0:1] + deg_ref[1, :, 0:1]
    neigh = acc / jnp.maximum(deg, 1.0)
    h = (jnp.dot(x, wst_ref[...], preferred_element_type=jnp.float32)
         + jnp.dot(neigh, wnt_ref[...], preferred_element_type=jnp.float32)
         + b_ref[...])
    o_ref[...] = jnp.maximum(h, 0.0)


_sage_dense = pl.pallas_call(
    _sage_dense_body,
    grid=(N // _RB,),
    in_specs=[
        pl.BlockSpec((_RB, D), lambda i: (i, 0)),
        pl.BlockSpec((NC, _RB, D), lambda i: (0, i, 0)),
        pl.BlockSpec((NC, _RB, 16), lambda i: (0, i, 0)),
        pl.BlockSpec((D, D), lambda i: (0, 0)),
        pl.BlockSpec((D, D), lambda i: (0, 0)),
        pl.BlockSpec((1, D), lambda i: (0, 0)),
    ],
    out_specs=pl.BlockSpec((_RB, D), lambda i: (i, 0)),
    out_shape=jax.ShapeDtypeStruct((N, D), jnp.float32),
)


def _final_body(h_ref, w1t_ref, b1_ref, w2t_ref, b2_ref, o_ref):
    h = jnp.maximum(
        jnp.dot(h_ref[...], w1t_ref[...], preferred_element_type=jnp.float32)
        + b1_ref[...], 0.0)
    o_ref[...] = (jnp.dot(h, w2t_ref[...], preferred_element_type=jnp.float32)
                  + b2_ref[...])


_final = pl.pallas_call(
    _final_body,
    grid=(N // _RB,),
    in_specs=[
        pl.BlockSpec((_RB, D), lambda i: (i, 0)),
        pl.BlockSpec((D, D), lambda i: (0, 0)),
        pl.BlockSpec((1, D), lambda i: (0, 0)),
        pl.BlockSpec((D, D), lambda i: (0, 0)),
        pl.BlockSpec((1, D), lambda i: (0, 0)),
    ],
    out_specs=pl.BlockSpec((_RB, D), lambda i: (i, 0)),
    out_shape=jax.ShapeDtypeStruct((N, D), jnp.float32),
)


def kernel(inputs, edge_index, ew, Ws1, Wn1, b1, Ws2, Wn2, b2, Ws3, Wn3, b3,
           lin1_W, lin1_b, lin2_W, lin2_b):
    src = edge_index[0]
    dst = edge_index[1]

    acc1, degf = _agg_deg(inputs, src, dst, ew)
    h = _sage_dense(inputs, acc1, degf, Ws1.T, Wn1.T, b1.reshape(1, D))
    acc2 = _agg(h, src, dst, ew)
    h = _sage_dense(h, acc2, degf, Ws2.T, Wn2.T, b2.reshape(1, D))
    acc3 = _agg(h, src, dst, ew)
    h = _sage_dense(h, acc3, degf, Ws3.T, Wn3.T, b3.reshape(1, D))
    h = _final(h, lin1_W.T, lin1_b.reshape(1, D), lin2_W.T, lin2_b.reshape(1, D))
    return h


# speedup vs baseline: 3.9283x; 3.9283x over previous
import functools

import jax
import jax.numpy as jnp
from jax import lax
from jax.experimental import pallas as pl
from jax.experimental.pallas import tpu as pltpu
from jax.experimental.pallas import tpu_sc as plsc

N = 10000
E = 320000
D = 128

NC = 2
NS = 16
NW = NC * NS
EPW = E // NW
CH = 80
NCH = EPW // CH
NP = 10240
RPT = NP // NS
ZR = 128


def _bcast_lane(vec16, r):
    idx = jnp.full((16, 1), r, dtype=jnp.int32)
    dn = lax.GatherDimensionNumbers(
        offset_dims=(), collapsed_slice_dims=(0,), start_index_map=(0,))
    return lax.gather(vec16, idx, dn, (1,),
                      mode=lax.GatherScatterMode.PROMISE_IN_BOUNDS)


def _make_agg(with_deg: bool):
    mesh = plsc.VectorSubcoreMesh(core_axis_name="c", subcore_axis_name="s")
    out_type = [jax.ShapeDtypeStruct((NC, NP, D), jnp.float32)]
    if with_deg:
        out_type.append(jax.ShapeDtypeStruct((NC, NP, 16), jnp.float32))
    scratch = [
        pltpu.VMEM((CH,), jnp.int32),
        pltpu.VMEM((CH,), jnp.int32),
        pltpu.VMEM((CH,), jnp.float32),
        pltpu.VMEM((CH, D), jnp.float32),
        pltpu.VMEM((ZR, D), jnp.float32),
        pltpu.VMEM_SHARED((NP, D), jnp.float32),
        pltpu.SemaphoreType.DMA,
    ]
    if with_deg:
        scratch += [
            pltpu.VMEM((CH, 16), jnp.float32),
            pltpu.VMEM((RPT, 16), jnp.float32),
            pltpu.VMEM_SHARED((NP, 16), jnp.float32),
        ]

    def body(x_hbm, src_hbm, dst_hbm, ew_hbm, *rest):
        if with_deg:
            (out_sum, out_deg, idx_v, dst_v, ew_v, rows_v, zbuf, acc_sh, sem,
             ones_v, zdeg, deg_sh) = rest
        else:
            (out_sum, idx_v, dst_v, ew_v, rows_v, zbuf, acc_sh, sem) = rest
        cid = lax.axis_index("c")
        sid = lax.axis_index("s")
        wid = sid * NC + cid
        rbase = sid * RPT

        def zrow(r, _):
            for c8 in range(D // 16):
                zbuf[r, pl.ds(c8 * 16, 16)] = jnp.zeros((16,), jnp.float32)
            return 0
        lax.fori_loop(0, ZR, zrow, 0)
        for k in range(RPT // ZR):
            pltpu.sync_copy(zbuf, acc_sh.at[pl.ds(rbase + k * ZR, ZR), :])
        if with_deg:
            def zdrow(r, _):
                zdeg[r, pl.ds(0, 16)] = jnp.zeros((16,), jnp.float32)
                return 0
            lax.fori_loop(0, RPT, zdrow, 0)

            def orow(r, _):
                ones_v[r, pl.ds(0, 16)] = jnp.ones((16,), jnp.float32)
                return 0
            lax.fori_loop(0, CH, orow, 0)
            pltpu.sync_copy(zdeg, deg_sh.at[pl.ds(rbase, RPT), :])
        plsc.subcore_barrier()

        def chunk(i, _):
            base = wid * EPW + i * CH
            pltpu.sync_copy(src_hbm.at[pl.ds(base, CH)], idx_v)
            pltpu.async_copy(x_hbm.at[idx_v], rows_v, sem).wait()
            pltpu.sync_copy(dst_hbm.at[pl.ds(base, CH)], dst_v)
            pltpu.sync_copy(ew_hbm.at[pl.ds(base, CH)], ew_v)

            def grp(j, _):
                ew16 = ew_v[pl.ds(j * 16, 16)]
                for r in range(16):
                    w = _bcast_lane(ew16, r)
                    e = j * 16 + r
                    for c8 in range(D // 16):
                        sl = pl.ds(c8 * 16, 16)
                        rows_v[e, sl] = rows_v[e, sl] * w
                return 0
            lax.fori_loop(0, CH // 16, grp, 0)

            pltpu.sync_copy(rows_v, acc_sh.at[dst_v], add=True)
            if with_deg:
                pltpu.sync_copy(ones_v, deg_sh.at[dst_v], add=True)
            return 0
        lax.fori_loop(0, NCH, chunk, 0)

        plsc.subcore_barrier()
        pltpu.sync_copy(acc_sh.at[pl.ds(rbase, RPT), :],
                        out_sum.at[cid, pl.ds(rbase, RPT), :])
        if with_deg:
            pltpu.sync_copy(deg_sh.at[pl.ds(rbase, RPT), :],
                            out_deg.at[cid, pl.ds(rbase, RPT), :])

    ot = tuple(out_type) if with_deg else out_type[0]
    cp = pltpu.CompilerParams(use_tc_tiling_on_sc=False)
    return pl.kernel(body, out_type=ot, mesh=mesh, scratch_types=scratch,
                     compiler_params=cp)


_agg_deg = _make_agg(with_deg=True)
_agg = _make_agg(with_deg=False)


_RB = 1024


def _sage_dense_body(x_ref, acc_ref, deg_ref, wst_ref, wnt_ref, b_ref, o_ref):
    x = x_ref[...]
    acc = acc_ref[0] + acc_ref[1]
    deg = deg_ref[0, :, 0:1] + deg_ref[1, :, 0:1]
    neigh = acc / jnp.maximum(deg, 1.0)
    h = (jnp.dot(x, wst_ref[...], preferred_element_type=jnp.float32)
         + jnp.dot(neigh, wnt_ref[...], preferred_element_type=jnp.float32)
         + b_ref[...])
    o_ref[...] = jnp.maximum(h, 0.0)


_sage_dense = pl.pallas_call(
    _sage_dense_body,
    grid=(NP // _RB,),
    in_specs=[
        pl.BlockSpec((_RB, D), lambda i: (i, 0)),
        pl.BlockSpec((NC, _RB, D), lambda i: (0, i, 0)),
        pl.BlockSpec((NC, _RB, 16), lambda i: (0, i, 0)),
        pl.BlockSpec((D, D), lambda i: (0, 0)),
        pl.BlockSpec((D, D), lambda i: (0, 0)),
        pl.BlockSpec((1, D), lambda i: (0, 0)),
    ],
    out_specs=pl.BlockSpec((_RB, D), lambda i: (i, 0)),
    out_shape=jax.ShapeDtypeStruct((NP, D), jnp.float32),
)


def _final_body(h_ref, w1t_ref, b1_ref, w2t_ref, b2_ref, o_ref):
    h = jnp.maximum(
        jnp.dot(h_ref[...], w1t_ref[...], preferred_element_type=jnp.float32)
        + b1_ref[...], 0.0)
    o_ref[...] = (jnp.dot(h, w2t_ref[...], preferred_element_type=jnp.float32)
                  + b2_ref[...])


_final = pl.pallas_call(
    _final_body,
    grid=(NP // _RB,),
    in_specs=[
        pl.BlockSpec((_RB, D), lambda i: (i, 0)),
        pl.BlockSpec((D, D), lambda i: (0, 0)),
        pl.BlockSpec((1, D), lambda i: (0, 0)),
        pl.BlockSpec((D, D), lambda i: (0, 0)),
        pl.BlockSpec((1, D), lambda i: (0, 0)),
    ],
    out_specs=pl.BlockSpec((_RB, D), lambda i: (i, 0)),
    out_shape=jax.ShapeDtypeStruct((NP, D), jnp.float32),
)


def kernel(inputs, edge_index, ew, Ws1, Wn1, b1, Ws2, Wn2, b2, Ws3, Wn3, b3,
           lin1_W, lin1_b, lin2_W, lin2_b):
    src = edge_index[0]
    dst = edge_index[1]
    xp = jnp.pad(inputs, ((0, NP - N), (0, 0)))

    acc1, degf = _agg_deg(xp, src, dst, ew)
    h = _sage_dense(xp, acc1, degf, Ws1.T, Wn1.T, b1.reshape(1, D))
    acc2 = _agg(h, src, dst, ew)
    h = _sage_dense(h, acc2, degf, Ws2.T, Wn2.T, b2.reshape(1, D))
    acc3 = _agg(h, src, dst, ew)
    h = _sage_dense(h, acc3, degf, Ws3.T, Wn3.T, b3.reshape(1, D))
    h = _final(h, lin1_W.T, lin1_b.reshape(1, D), lin2_W.T, lin2_b.reshape(1, D))
    return h[:N]
